# baseline (speedup 1.0000x reference)
import functools

import jax
import jax.numpy as jnp
from jax import lax
from jax.experimental import pallas as pl
from jax.experimental.pallas import tpu as pltpu
from jax.experimental.pallas import tpu_sc as plsc

N = 6_400_000
S = 100_000
SP = 106_496
B = 4000
NBLK = N // B
NW = 32
NIT = NBLK // NW
STRIPE = SP // 16
CB = B + 160


def _sc_body(qa_hbm, rx_hbm, ry_hbm, rz_hbm, seg_hbm, out_hbm,
             qa_v0, rx_v0, ry_v0, rz_v0, seg_v0,
             qa_v1, rx_v1, ry_v1, rz_v1, seg_v1, cs_v,
             cseg_v, cpx_v, cpy_v, cpz_v, zb_v,
             acc_x, acc_y, acc_z, sem, sem_in):
    c = lax.axis_index("c")
    s = lax.axis_index("s")
    wid = c * 16 + s
    accs = (acc_x, acc_y, acc_z)
    banks = ((qa_v0, rx_v0, ry_v0, rz_v0, seg_v0),
             (qa_v1, rx_v1, ry_v1, rz_v1, seg_v1))
    cps = (cpx_v, cpy_v, cpz_v)
    r_hbms = (rx_hbm, ry_hbm, rz_hbm)

    iota = lax.iota(jnp.int32, 16)
    zero16 = jnp.zeros((16,), jnp.float32)
    neg1_16 = jnp.full((16,), -1, jnp.int32)

    def _zb(k, _):
        zb_v[pl.ds(k * 16, 16)] = zero16
        return 0

    lax.fori_loop(0, STRIPE // 16, _zb, 0)
    for a in accs:
        pltpu.sync_copy(zb_v, a.at[pl.ds(s * STRIPE, STRIPE)])
    plsc.subcore_barrier()

    for bank in banks:
        bank[4][pl.ds(0, 16)] = neg1_16
        bank[4][pl.ds(B + 16, 16)] = neg1_16

    dump16 = S + wid * 16 + iota

    def _issue_inputs(blk, bank):
        base = blk * B
        qa_b, rx_b, ry_b, rz_b, seg_b = bank
        pltpu.async_copy(qa_hbm.at[pl.ds(base, B)], qa_b, sem_in)
        pltpu.async_copy(seg_hbm.at[pl.ds(base, B)],
                         seg_b.at[pl.ds(16, B)], sem_in)
        for t in range(3):
            pltpu.async_copy(r_hbms[t].at[pl.ds(base, B)],
                             (rx_b, ry_b, rz_b)[t], sem_in)

    def _wait_inputs(bank):
        qa_b, rx_b, ry_b, rz_b, seg_b = bank
        base0 = 0
        pltpu.make_async_copy(qa_hbm.at[pl.ds(base0, B)], qa_b, sem_in).wait()
        pltpu.make_async_copy(seg_hbm.at[pl.ds(base0, B)],
                              seg_b.at[pl.ds(16, B)], sem_in).wait()
        for t in range(3):
            pltpu.make_async_copy(r_hbms[t].at[pl.ds(base0, B)],
                                  (rx_b, ry_b, rz_b)[t], sem_in).wait()

    _issue_inputs(wid, banks[0])

    def _phase(i, prev_trips, bank, nbank):
        qa_v, rx_v, ry_v, rz_v, seg_v = bank
        rs = (rx_v, ry_v, rz_v)

        _wait_inputs(bank)

        @pl.when(i + 1 < NIT)
        def _():
            _issue_inputs(wid + NW * (i + 1), nbank)

        def _drain(k, _):
            for t, a in enumerate(accs):
                pltpu.make_async_copy(
                    cps[t].at[pl.ds(k * 128, 128)],
                    a.at[cseg_v.at[pl.ds(k * 128, 128)]],
                    sem,
                ).wait()
            return 0

        lax.fori_loop(0, prev_trips, _drain, 0)

        @plsc.parallel_loop(0, B // 16, unroll=10, carry=neg1_16)
        def cursor(j, cursor):
            jb = j * 16
            csbase = j * 48
            q16 = qa_v[pl.ds(jb, 16)]
            seg = seg_v[pl.ds(jb + 16, 16)]
            segn = plsc.load_gather(seg_v, [iota + (jb + 17)])
            segp = plsc.load_gather(seg_v, [iota + (jb + 15)])
            end = jnp.logical_or(seg != segn, iota == 15)
            sp = seg != segp
            rank = plsc.cumsum(jnp.where(end, 1, 0))
            pe = plsc.cummax(jnp.where(sp, iota - 1, -1))
            pe_ok = pe >= 0
            pec = jnp.maximum(pe, 0) + csbase
            rsum = []
            for t in range(3):
                cs = plsc.cumsum(q16 * rs[t][pl.ds(jb, 16)])
                cs_v[pl.ds(csbase + t * 16, 16)] = cs
                g = plsc.load_gather(cs_v, [pec + t * 16])
                rsum.append(cs - jnp.where(pe_ok, g, 0.0))
            idxv = cursor + rank
            plsc.store_scatter(cseg_v, [idxv], seg, mask=end)
            for t in range(3):
                plsc.store_scatter(cps[t], [idxv], rsum[t], mask=end)
            nst = plsc.all_reduce_population_count(end)
            return cursor + nst
        count = jnp.max(cursor) + 1
        fl = (count // 16) * 16
        for k in range(10):
            pos = fl + 16 * k + iota
            plsc.store_scatter(cseg_v, [pos], dump16, mask=pos >= count)
        trips = (count + 127) // 128

        def _scat(k, _):
            for t, a in enumerate(accs):
                pltpu.async_copy(
                    cps[t].at[pl.ds(k * 128, 128)],
                    a.at[cseg_v.at[pl.ds(k * 128, 128)]],
                    sem,
                    add=True,
                )
            return 0

        lax.fori_loop(0, trips, _scat, 0)
        return trips

    def _block(d, prev_trips):
        t0 = _phase(2 * d, prev_trips, banks[0], banks[1])
        return _phase(2 * d + 1, t0, banks[1], banks[0])

    final_trips = lax.fori_loop(0, NIT // 2, _block, jnp.int32(0))

    def _drain_last(k, _):
        for t, a in enumerate(accs):
            pltpu.make_async_copy(
                cps[t].at[pl.ds(k * 128, 128)],
                a.at[cseg_v.at[pl.ds(k * 128, 128)]],
                sem,
            ).wait()
        return 0

    lax.fori_loop(0, final_trips, _drain_last, 0)
    plsc.subcore_barrier()
    for t, a in enumerate(accs):
        pltpu.sync_copy(a.at[pl.ds(s * STRIPE, STRIPE)],
                        out_hbm.at[pl.ds((c * 3 + t) * SP + s * STRIPE, STRIPE)])


_sc_call = functools.partial(
    pl.kernel,
    out_type=jax.ShapeDtypeStruct((6 * SP,), jnp.float32),
    mesh=plsc.VectorSubcoreMesh(core_axis_name="c", subcore_axis_name="s"),
    compiler_params=pltpu.CompilerParams(needs_layout_passes=False),
    scratch_types=[
        pltpu.VMEM((B,), jnp.float32),
        pltpu.VMEM((B,), jnp.float32),
        pltpu.VMEM((B,), jnp.float32),
        pltpu.VMEM((B,), jnp.float32),
        pltpu.VMEM((B + 32,), jnp.int32),
        pltpu.VMEM((B,), jnp.float32),
        pltpu.VMEM((B,), jnp.float32),
        pltpu.VMEM((B,), jnp.float32),
        pltpu.VMEM((B,), jnp.float32),
        pltpu.VMEM((B + 32,), jnp.int32),
        pltpu.VMEM((B // 16 * 48,), jnp.float32),
        pltpu.VMEM((CB,), jnp.int32),
        pltpu.VMEM((CB,), jnp.float32),
        pltpu.VMEM((CB,), jnp.float32),
        pltpu.VMEM((CB,), jnp.float32),
        pltpu.VMEM((STRIPE,), jnp.float32),
        pltpu.VMEM_SHARED((SP,), jnp.float32),
        pltpu.VMEM_SHARED((SP,), jnp.float32),
        pltpu.VMEM_SHARED((SP,), jnp.float32),
        pltpu.SemaphoreType.DMA,
        pltpu.SemaphoreType.DMA,
    ],
)(_sc_body)


def _combine_body(a_ref, o_ref):
    o_ref[...] = a_ref[0] + a_ref[1]


_combine = pl.pallas_call(
    _combine_body,
    out_shape=jax.ShapeDtypeStruct((3, SP), jnp.float32),
)


def kernel(Qa, Ra, batch_seg):
    seg32 = batch_seg.astype(jnp.int32)
    rx, ry, rz = Ra[:, 0], Ra[:, 1], Ra[:, 2]
    partial = _sc_call(Qa, rx, ry, rz, seg32)
    out = _combine(partial.reshape(2, 3, SP))
    return out[:, :S].T


# speedup vs baseline: 1.9308x; 1.9308x over previous
import functools

import jax
import jax.numpy as jnp
from jax import lax
from jax.experimental import pallas as pl
from jax.experimental.pallas import tpu as pltpu
from jax.experimental.pallas import tpu_sc as plsc

N = 6_400_000
S = 100_000
SP = 106_496
B = 4000
NBLK = N // B
NW = 32
NIT = NBLK // NW
STRIPE = SP // 16
CB = B + 160


def _sc_body(qa_hbm, rx_hbm, ry_hbm, rz_hbm, seg_hbm, out_hbm,
             qa_v0, rx_v0, ry_v0, rz_v0, seg_v0,
             qa_v1, rx_v1, ry_v1, rz_v1, seg_v1, cs_v,
             cseg_v, cpx_v, cpy_v, cpz_v, zb_v,
             acc_x, acc_y, acc_z, sem, sem_in):
    c = lax.axis_index("c")
    s = lax.axis_index("s")
    wid = c * 16 + s
    accs = (acc_x, acc_y, acc_z)
    banks = ((qa_v0, rx_v0, ry_v0, rz_v0, seg_v0),
             (qa_v1, rx_v1, ry_v1, rz_v1, seg_v1))
    cps = (cpx_v, cpy_v, cpz_v)
    r_hbms = (rx_hbm, ry_hbm, rz_hbm)

    iota = lax.iota(jnp.int32, 16)
    zero16 = jnp.zeros((16,), jnp.float32)
    neg1_16 = jnp.full((16,), -1, jnp.int32)

    def _zb(k, _):
        zb_v[pl.ds(k * 16, 16)] = zero16
        return 0

    lax.fori_loop(0, STRIPE // 16, _zb, 0)
    for a in accs:
        pltpu.sync_copy(zb_v, a.at[pl.ds(s * STRIPE, STRIPE)])
    plsc.subcore_barrier()

    for bank in banks:
        bank[4][pl.ds(0, 16)] = neg1_16
        bank[4][pl.ds(B + 16, 16)] = neg1_16

    dump16 = S + wid * 16 + iota

    def _issue_inputs(blk, bank):
        base = blk * B
        qa_b, rx_b, ry_b, rz_b, seg_b = bank
        pltpu.async_copy(qa_hbm.at[pl.ds(base, B)], qa_b, sem_in)
        pltpu.async_copy(seg_hbm.at[pl.ds(base, B)],
                         seg_b.at[pl.ds(16, B)], sem_in)
        for t in range(3):
            pltpu.async_copy(r_hbms[t].at[pl.ds(base, B)],
                             (rx_b, ry_b, rz_b)[t], sem_in)

    def _wait_inputs(bank):
        qa_b, rx_b, ry_b, rz_b, seg_b = bank
        base0 = 0
        pltpu.make_async_copy(qa_hbm.at[pl.ds(base0, B)], qa_b, sem_in).wait()
        pltpu.make_async_copy(seg_hbm.at[pl.ds(base0, B)],
                              seg_b.at[pl.ds(16, B)], sem_in).wait()
        for t in range(3):
            pltpu.make_async_copy(r_hbms[t].at[pl.ds(base0, B)],
                                  (rx_b, ry_b, rz_b)[t], sem_in).wait()

    _issue_inputs(wid, banks[0])

    def _phase(i, prev_trips, bank, nbank):
        qa_v, rx_v, ry_v, rz_v, seg_v = bank
        rs = (rx_v, ry_v, rz_v)

        _wait_inputs(bank)

        @pl.when(i + 1 < NIT)
        def _():
            _issue_inputs(wid + NW * (i + 1), nbank)

        def _drain(k, _):
            for t, a in enumerate(accs):
                pltpu.make_async_copy(
                    cps[t].at[pl.ds(k * 128, 128)],
                    a.at[cseg_v.at[pl.ds(k * 128, 128)]],
                    sem,
                ).wait()
            return 0

        lax.fori_loop(0, prev_trips, _drain, 0)

        @plsc.parallel_loop(0, B // 16, unroll=5, carry=neg1_16)
        def cursor(j, cursor):
            jb = j * 16
            csbase = j * 48
            q16 = qa_v[pl.ds(jb, 16)]
            seg = seg_v[pl.ds(jb + 16, 16)]
            segn = plsc.load_gather(seg_v, [iota + (jb + 17)])
            segp = plsc.load_gather(seg_v, [iota + (jb + 15)])
            end = jnp.logical_or(seg != segn, iota == 15)
            sp = seg != segp
            rank = plsc.cumsum(jnp.where(end, 1, 0))
            pe = plsc.cummax(jnp.where(sp, iota - 1, -1))
            pe_ok = pe >= 0
            pec = jnp.maximum(pe, 0) + csbase
            rsum = []
            for t in range(3):
                cs = plsc.cumsum(q16 * rs[t][pl.ds(jb, 16)])
                cs_v[pl.ds(csbase + t * 16, 16)] = cs
                g = plsc.load_gather(cs_v, [pec + t * 16])
                rsum.append(cs - jnp.where(pe_ok, g, 0.0))
            idxv = cursor + rank
            plsc.store_scatter(cseg_v, [idxv], seg, mask=end)
            for t in range(3):
                plsc.store_scatter(cps[t], [idxv], rsum[t], mask=end)
            nst = plsc.all_reduce_population_count(end)
            return cursor + nst
        count = jnp.max(cursor) + 1
        fl = (count // 16) * 16
        for k in range(10):
            pos = fl + 16 * k + iota
            plsc.store_scatter(cseg_v, [pos], dump16, mask=pos >= count)
        trips = (count + 127) // 128

        def _scat(k, _):
            for t, a in enumerate(accs):
                pltpu.async_copy(
                    cps[t].at[pl.ds(k * 128, 128)],
                    a.at[cseg_v.at[pl.ds(k * 128, 128)]],
                    sem,
                    add=True,
                )
            return 0

        lax.fori_loop(0, trips, _scat, 0)
        return trips

    def _block(d, prev_trips):
        t0 = _phase(2 * d, prev_trips, banks[0], banks[1])
        return _phase(2 * d + 1, t0, banks[1], banks[0])

    final_trips = lax.fori_loop(0, NIT // 2, _block, jnp.int32(0))

    def _drain_last(k, _):
        for t, a in enumerate(accs):
            pltpu.make_async_copy(
                cps[t].at[pl.ds(k * 128, 128)],
                a.at[cseg_v.at[pl.ds(k * 128, 128)]],
                sem,
            ).wait()
        return 0

    lax.fori_loop(0, final_trips, _drain_last, 0)
    plsc.subcore_barrier()
    for t, a in enumerate(accs):
        pltpu.sync_copy(a.at[pl.ds(s * STRIPE, STRIPE)],
                        out_hbm.at[pl.ds((c * 3 + t) * SP + s * STRIPE, STRIPE)])


_sc_call = functools.partial(
    pl.kernel,
    out_type=jax.ShapeDtypeStruct((6 * SP,), jnp.float32),
    mesh=plsc.VectorSubcoreMesh(core_axis_name="c", subcore_axis_name="s"),
    compiler_params=pltpu.CompilerParams(needs_layout_passes=False),
    scratch_types=[
        pltpu.VMEM((B,), jnp.float32),
        pltpu.VMEM((B,), jnp.float32),
        pltpu.VMEM((B,), jnp.float32),
        pltpu.VMEM((B,), jnp.float32),
        pltpu.VMEM((B + 32,), jnp.int32),
        pltpu.VMEM((B,), jnp.float32),
        pltpu.VMEM((B,), jnp.float32),
        pltpu.VMEM((B,), jnp.float32),
        pltpu.VMEM((B,), jnp.float32),
        pltpu.VMEM((B + 32,), jnp.int32),
        pltpu.VMEM((B // 16 * 48,), jnp.float32),
        pltpu.VMEM((CB,), jnp.int32),
        pltpu.VMEM((CB,), jnp.float32),
        pltpu.VMEM((CB,), jnp.float32),
        pltpu.VMEM((CB,), jnp.float32),
        pltpu.VMEM((STRIPE,), jnp.float32),
        pltpu.VMEM_SHARED((SP,), jnp.float32),
        pltpu.VMEM_SHARED((SP,), jnp.float32),
        pltpu.VMEM_SHARED((SP,), jnp.float32),
        pltpu.SemaphoreType.DMA,
        pltpu.SemaphoreType.DMA,
    ],
)(_sc_body)


def _combine_body(a_ref, o_ref):
    o_ref[...] = a_ref[0] + a_ref[1]


_combine = pl.pallas_call(
    _combine_body,
    out_shape=jax.ShapeDtypeStruct((3, SP), jnp.float32),
)


def kernel(Qa, Ra, batch_seg):
    seg32 = batch_seg.astype(jnp.int32)
    rx, ry, rz = Ra[:, 0], Ra[:, 1], Ra[:, 2]
    partial = _sc_call(Qa, rx, ry, rz, seg32)
    out = _combine(partial.reshape(2, 3, SP))
    return out[:, :S].T
